# baseline (speedup 1.0000x reference)
import functools

import jax
import jax.numpy as jnp
from jax import lax
from jax.experimental import pallas as pl
from jax.experimental.pallas import tpu as pltpu
from jax.experimental.pallas import tpu_sc as plsc

D = 256
DW = D // 2
S = 16
EMB = 256
NC = 2
NS = 16
NW = NC * NS
BP = 10240
SEEDS_PER_W = BP // NW
CS = 16
NCHUNK = SEEDS_PER_W // CS
NBUF = 2
TB = 2048
PACK_ROWS = 2000


def _rne16(f):
    u = lax.bitcast_convert_type(f, jnp.int32)
    odd = lax.bitwise_and(
        lax.shift_right_logical(u, jnp.int32(16)), jnp.int32(1))
    r = lax.shift_right_logical(u + jnp.int32(32767) + odd, jnp.int32(16))
    return lax.bitwise_and(r, jnp.int32(65535))


def _lo_f32(x):
    return lax.bitcast_convert_type(lax.shift_left(x, jnp.int32(16)),
                                    jnp.float32)


def _hi_f32(x):
    return lax.bitcast_convert_type(lax.bitwise_and(x, jnp.int32(-65536)),
                                    jnp.float32)


def _pack_body(f_ref, o_ref):
    x = f_ref[...]
    lo = _rne16(x[:, :DW])
    hi = _rne16(x[:, DW:])
    o_ref[...] = lax.bitwise_or(lax.shift_left(hi, jnp.int32(16)), lo)


def _make_sc_gather_sum():
    mesh = plsc.VectorSubcoreMesh(core_axis_name="c", subcore_axis_name="s")

    @functools.partial(
        pl.kernel,
        mesh=mesh,
        out_type=jax.ShapeDtypeStruct((BP, 2 * DW), jnp.int32),
        scratch_types=(
            [pltpu.VMEM((SEEDS_PER_W * S,), jnp.int32),
             pltpu.VMEM((SEEDS_PER_W,), jnp.int32)]
            + [pltpu.VMEM((CS * S, DW), jnp.int32)] * NBUF
            + [pltpu.VMEM((CS, DW), jnp.int32)] * NBUF
            + [pltpu.VMEM((CS, 2 * DW), jnp.int32)] * NBUF
            + [pltpu.SemaphoreType.DMA] * (2 * NBUF)
        ),
    )
    def sc_gather_sum(feat_hbm, nodes_hbm, neigh_hbm, comb_out,
                      nidx_v, sidx_v, *bufs):
        nbufs = bufs[0:NBUF]
        sbufs = bufs[NBUF:2 * NBUF]
        obufs = bufs[2 * NBUF:3 * NBUF]
        gsems = bufs[3 * NBUF:4 * NBUF]
        osems = bufs[4 * NBUF:5 * NBUF]
        wid = lax.axis_index("s") * NC + lax.axis_index("c")
        base = pl.multiple_of(wid * SEEDS_PER_W, SEEDS_PER_W)
        pltpu.sync_copy(neigh_hbm.at[pl.ds(base * S, SEEDS_PER_W * S)], nidx_v)
        pltpu.sync_copy(nodes_hbm.at[pl.ds(base, SEEDS_PER_W)], sidx_v)

        def fire_gather(g, b):
            off_n = pl.multiple_of(g * (CS * S), CS * S)
            off_s = pl.multiple_of(g * CS, CS)
            half = CS * S // 2
            pltpu.async_copy(
                feat_hbm.at[nidx_v.at[pl.ds(off_n, half)]],
                nbufs[b].at[pl.ds(0, half)], gsems[b])
            pltpu.async_copy(
                feat_hbm.at[nidx_v.at[pl.ds(off_n + half, half)]],
                nbufs[b].at[pl.ds(half, half)], gsems[b])
            pltpu.async_copy(
                feat_hbm.at[sidx_v.at[pl.ds(off_s, CS)]], sbufs[b], gsems[b])

        def wait_gather(b):
            pltpu.make_async_copy(
                feat_hbm.at[pl.ds(0, CS * S)], nbufs[b], gsems[b]).wait()
            pltpu.make_async_copy(
                feat_hbm.at[pl.ds(0, CS)], sbufs[b], gsems[b]).wait()

        def fire_out(g, b):
            row = pl.multiple_of(base + g * CS, CS)
            pltpu.async_copy(obufs[b], comb_out.at[pl.ds(row, CS)], osems[b])

        def drain_out(b):
            pltpu.make_async_copy(
                obufs[b], comb_out.at[pl.ds(0, CS)], osems[b]).wait()

        def compute(b):
            nb, sb, ob = nbufs[b], sbufs[b], obufs[b]

            def seed_body(s0, _):
                r0 = s0 * S
                for v in range(DW // 16):
                    x = nb[r0, pl.ds(v * 16, 16)]
                    a_lo, a_hi = _lo_f32(x), _hi_f32(x)
                    for r in range(1, S):
                        y = nb[r0 + r, pl.ds(v * 16, 16)]
                        a_lo = a_lo + _lo_f32(y)
                        a_hi = a_hi + _hi_f32(y)
                    word = lax.bitwise_or(
                        lax.shift_left(_rne16(a_hi), jnp.int32(16)),
                        _rne16(a_lo))
                    ob[s0, pl.ds(DW + v * 16, 16)] = word
                    ob[s0, pl.ds(v * 16, 16)] = sb[s0, pl.ds(v * 16, 16)]
                return 0

            lax.fori_loop(0, CS, seed_body, 0, unroll=False)

        for b in range(NBUF):
            fire_gather(b, b)

        def ring_body(p, _):
            for b in range(NBUF):
                g = p * NBUF + b
                wait_gather(b)
                compute(b)

                @pl.when(p > 0)
                def _():
                    drain_out(b)

                fire_out(g, b)

                @pl.when(g + NBUF < NCHUNK)
                def _():
                    fire_gather(g + NBUF, b)
            return 0

        lax.fori_loop(0, NCHUNK // NBUF, ring_body, 0, unroll=False)
        for b in range(NBUF):
            drain_out(b)

    return sc_gather_sum


_sc_gather_sum = _make_sc_gather_sum()


def _mm_body(w_ref, c_ref, o_ref):
    w = w_ref[...]
    cw = c_ref[...]
    sw = cw[:, :DW]
    mw = cw[:, DW:]
    scale = jnp.float32(1.0 / S)
    dn = (((1,), (1,)), ((), ()))
    acc = lax.dot_general(w[:, 0 * DW:1 * DW], _lo_f32(sw), dn,
                          preferred_element_type=jnp.float32)
    acc += lax.dot_general(w[:, 1 * DW:2 * DW], _hi_f32(sw), dn,
                           preferred_element_type=jnp.float32)
    acc += lax.dot_general(w[:, 2 * DW:3 * DW], _lo_f32(mw) * scale, dn,
                           preferred_element_type=jnp.float32)
    acc += lax.dot_general(w[:, 3 * DW:4 * DW], _hi_f32(mw) * scale, dn,
                           preferred_element_type=jnp.float32)
    o_ref[...] = jnp.maximum(acc, 0.0)


def kernel(nodes, neigh_idx, features, weight):
    batch = nodes.shape[0]
    pad = BP - batch
    nodes_p = jnp.concatenate(
        [nodes.astype(jnp.int32), jnp.zeros((pad,), jnp.int32)])
    neigh_p = jnp.concatenate(
        [neigh_idx.astype(jnp.int32).reshape(-1),
         jnp.zeros((pad * S,), jnp.int32)])

    n_nodes = features.shape[0]
    feat_packed = pl.pallas_call(
        _pack_body,
        grid=(n_nodes // PACK_ROWS,),
        in_specs=[pl.BlockSpec((PACK_ROWS, D), lambda i: (i, 0))],
        out_specs=pl.BlockSpec((PACK_ROWS, DW), lambda i: (i, 0)),
        out_shape=jax.ShapeDtypeStruct((n_nodes, DW), jnp.int32),
    )(features)

    comb_i32 = _sc_gather_sum(feat_packed, nodes_p, neigh_p)

    out_full = pl.pallas_call(
        _mm_body,
        grid=(BP // TB,),
        in_specs=[
            pl.BlockSpec((EMB, 2 * D), lambda i: (0, 0)),
            pl.BlockSpec((TB, 2 * DW), lambda i: (i, 0)),
        ],
        out_specs=pl.BlockSpec((EMB, TB), lambda i: (0, i)),
        out_shape=jax.ShapeDtypeStruct((EMB, BP), jnp.float32),
    )(weight, comb_i32)
    return out_full[:, :batch]


# speedup vs baseline: 1.0046x; 1.0046x over previous
import functools

import jax
import jax.numpy as jnp
from jax import lax
from jax.experimental import pallas as pl
from jax.experimental.pallas import tpu as pltpu
from jax.experimental.pallas import tpu_sc as plsc

D = 256
DW = D // 2
S = 16
EMB = 256
NC = 2
NS = 16
NW = NC * NS
BP = 10240
SEEDS_PER_W = BP // NW
CS = 8
NCHUNK = SEEDS_PER_W // CS
NBUF = 4
TB = 2048
PACK_ROWS = 2000


def _rne16(f):
    u = lax.bitcast_convert_type(f, jnp.int32)
    odd = lax.bitwise_and(
        lax.shift_right_logical(u, jnp.int32(16)), jnp.int32(1))
    r = lax.shift_right_logical(u + jnp.int32(32767) + odd, jnp.int32(16))
    return lax.bitwise_and(r, jnp.int32(65535))


def _lo_f32(x):
    return lax.bitcast_convert_type(lax.shift_left(x, jnp.int32(16)),
                                    jnp.float32)


def _hi_f32(x):
    return lax.bitcast_convert_type(lax.bitwise_and(x, jnp.int32(-65536)),
                                    jnp.float32)


def _pack_body(f_ref, o_ref):
    x = f_ref[...]
    lo = _rne16(x[:, :DW])
    hi = _rne16(x[:, DW:])
    o_ref[...] = lax.bitwise_or(lax.shift_left(hi, jnp.int32(16)), lo)


def _make_sc_gather_sum():
    mesh = plsc.VectorSubcoreMesh(core_axis_name="c", subcore_axis_name="s")

    @functools.partial(
        pl.kernel,
        mesh=mesh,
        out_type=jax.ShapeDtypeStruct((BP, 2 * DW), jnp.int32),
        scratch_types=(
            [pltpu.VMEM((SEEDS_PER_W * S,), jnp.int32),
             pltpu.VMEM((SEEDS_PER_W,), jnp.int32)]
            + [pltpu.VMEM((CS * S, DW), jnp.int32)] * NBUF
            + [pltpu.VMEM((CS, DW), jnp.int32)] * NBUF
            + [pltpu.VMEM((CS, 2 * DW), jnp.int32)] * NBUF
            + [pltpu.SemaphoreType.DMA] * (2 * NBUF)
        ),
    )
    def sc_gather_sum(feat_hbm, nodes_hbm, neigh_hbm, comb_out,
                      nidx_v, sidx_v, *bufs):
        nbufs = bufs[0:NBUF]
        sbufs = bufs[NBUF:2 * NBUF]
        obufs = bufs[2 * NBUF:3 * NBUF]
        gsems = bufs[3 * NBUF:4 * NBUF]
        osems = bufs[4 * NBUF:5 * NBUF]
        wid = lax.axis_index("s") * NC + lax.axis_index("c")
        base = pl.multiple_of(wid * SEEDS_PER_W, SEEDS_PER_W)
        pltpu.sync_copy(neigh_hbm.at[pl.ds(base * S, SEEDS_PER_W * S)], nidx_v)
        pltpu.sync_copy(nodes_hbm.at[pl.ds(base, SEEDS_PER_W)], sidx_v)

        def fire_gather(g, b):
            off_n = pl.multiple_of(g * (CS * S), CS * S)
            off_s = pl.multiple_of(g * CS, CS)
            pltpu.async_copy(
                feat_hbm.at[nidx_v.at[pl.ds(off_n, CS * S)]], nbufs[b], gsems[b])
            pltpu.async_copy(
                feat_hbm.at[sidx_v.at[pl.ds(off_s, CS)]], sbufs[b], gsems[b])

        def wait_gather(b):
            pltpu.make_async_copy(
                feat_hbm.at[pl.ds(0, CS * S)], nbufs[b], gsems[b]).wait()
            pltpu.make_async_copy(
                feat_hbm.at[pl.ds(0, CS)], sbufs[b], gsems[b]).wait()

        def fire_out(g, b):
            row = pl.multiple_of(base + g * CS, CS)
            pltpu.async_copy(obufs[b], comb_out.at[pl.ds(row, CS)], osems[b])

        def drain_out(b):
            pltpu.make_async_copy(
                obufs[b], comb_out.at[pl.ds(0, CS)], osems[b]).wait()

        def compute(b):
            nb, sb, ob = nbufs[b], sbufs[b], obufs[b]

            def seed_body(s0, _):
                r0 = s0 * S
                for v in range(DW // 16):
                    x = nb[r0, pl.ds(v * 16, 16)]
                    a_lo, a_hi = _lo_f32(x), _hi_f32(x)
                    for r in range(1, S):
                        y = nb[r0 + r, pl.ds(v * 16, 16)]
                        a_lo = a_lo + _lo_f32(y)
                        a_hi = a_hi + _hi_f32(y)
                    word = lax.bitwise_or(
                        lax.shift_left(_rne16(a_hi), jnp.int32(16)),
                        _rne16(a_lo))
                    ob[s0, pl.ds(DW + v * 16, 16)] = word
                    ob[s0, pl.ds(v * 16, 16)] = sb[s0, pl.ds(v * 16, 16)]
                return 0

            lax.fori_loop(0, CS, seed_body, 0, unroll=False)

        for b in range(NBUF):
            fire_gather(b, b)

        def ring_body(p, _):
            for b in range(NBUF):
                g = p * NBUF + b
                wait_gather(b)
                compute(b)

                @pl.when(p > 0)
                def _():
                    drain_out(b)

                fire_out(g, b)

                @pl.when(g + NBUF < NCHUNK)
                def _():
                    fire_gather(g + NBUF, b)
            return 0

        lax.fori_loop(0, NCHUNK // NBUF, ring_body, 0, unroll=False)
        for b in range(NBUF):
            drain_out(b)

    return sc_gather_sum


_sc_gather_sum = _make_sc_gather_sum()


def _mm_body(w_ref, c_ref, o_ref):
    w = w_ref[...]
    cw = c_ref[...]
    sw = cw[:, :DW]
    mw = cw[:, DW:]
    scale = jnp.float32(1.0 / S)
    dn = (((1,), (1,)), ((), ()))
    acc = lax.dot_general(w[:, 0 * DW:1 * DW], _lo_f32(sw), dn,
                          preferred_element_type=jnp.float32)
    acc += lax.dot_general(w[:, 1 * DW:2 * DW], _hi_f32(sw), dn,
                           preferred_element_type=jnp.float32)
    acc += lax.dot_general(w[:, 2 * DW:3 * DW], _lo_f32(mw) * scale, dn,
                           preferred_element_type=jnp.float32)
    acc += lax.dot_general(w[:, 3 * DW:4 * DW], _hi_f32(mw) * scale, dn,
                           preferred_element_type=jnp.float32)
    o_ref[...] = jnp.maximum(acc, 0.0)


def kernel(nodes, neigh_idx, features, weight):
    batch = nodes.shape[0]
    pad = BP - batch
    nodes_p = jnp.concatenate(
        [nodes.astype(jnp.int32), jnp.zeros((pad,), jnp.int32)])
    neigh_p = jnp.concatenate(
        [neigh_idx.astype(jnp.int32).reshape(-1),
         jnp.zeros((pad * S,), jnp.int32)])

    n_nodes = features.shape[0]
    feat_packed = pl.pallas_call(
        _pack_body,
        grid=(n_nodes // PACK_ROWS,),
        in_specs=[pl.BlockSpec((PACK_ROWS, D), lambda i: (i, 0))],
        out_specs=pl.BlockSpec((PACK_ROWS, DW), lambda i: (i, 0)),
        out_shape=jax.ShapeDtypeStruct((n_nodes, DW), jnp.int32),
    )(features)

    comb_i32 = _sc_gather_sum(feat_packed, nodes_p, neigh_p)

    out_full = pl.pallas_call(
        _mm_body,
        grid=(BP // TB,),
        in_specs=[
            pl.BlockSpec((EMB, 2 * D), lambda i: (0, 0)),
            pl.BlockSpec((TB, 2 * DW), lambda i: (i, 0)),
        ],
        out_specs=pl.BlockSpec((EMB, TB), lambda i: (0, i)),
        out_shape=jax.ShapeDtypeStruct((EMB, BP), jnp.float32),
    )(weight, comb_i32)
    return out_full[:, :batch]
